# baseline (speedup 1.0000x reference)
import functools

import jax
import jax.numpy as jnp
from jax import lax
from jax.experimental import pallas as pl
from jax.experimental.pallas import tpu as pltpu
from jax.experimental.pallas import tpu_sc as plsc

D = 128
ROW_BLK = 1024
NC = 2
NS = 16
NW = NC * NS
WIN = 128


def _sc_degree(n_pad, wpw):
    rows_per_sub = n_pad // NS
    mesh = plsc.VectorSubcoreMesh(core_axis_name="c", subcore_axis_name="s")

    @functools.partial(
        pl.kernel,
        mesh=mesh,
        out_type=jax.ShapeDtypeStruct((NC, n_pad), jnp.float32),
        scratch_types=[
            pltpu.VMEM((wpw, WIN), jnp.int32),
            pltpu.VMEM((WIN,), jnp.float32),
            pltpu.VMEM((n_pad // NS,), jnp.float32),
            pltpu.VMEM_SHARED((n_pad,), jnp.float32),
        ],
    )
    def deg_kernel(dst_hbm, out_hbm, idx_v, ones_v, zeros_v, acc):
        cid = lax.axis_index("c")
        sid = lax.axis_index("s")
        wid = sid * NC + cid
        one = jnp.full((16,), 1.0, dtype=jnp.float32)
        zero = jnp.zeros((16,), dtype=jnp.float32)
        for i in range(WIN // 16):
            ones_v[pl.ds(i * 16, 16)] = one

        def fill_zero(i, _):
            zeros_v[pl.ds(i * 16, 16)] = zero
            return _

        lax.fori_loop(0, rows_per_sub // 16, fill_zero, None)
        pltpu.sync_copy(zeros_v, acc.at[pl.ds(sid * rows_per_sub, rows_per_sub)])
        pltpu.sync_copy(dst_hbm.at[wid], idx_v)
        plsc.subcore_barrier()

        def body(j, _):
            pltpu.sync_copy(ones_v, acc.at[idx_v.at[j]], add=True)
            return _

        lax.fori_loop(0, wpw, body, None)
        plsc.subcore_barrier()
        pltpu.sync_copy(
            acc.at[pl.ds(sid * rows_per_sub, rows_per_sub)],
            out_hbm.at[cid, pl.ds(sid * rows_per_sub, rows_per_sub)],
        )

    return deg_kernel


def _sc_aggregate(n_pad, wpw):
    rows_per_sub = n_pad // NS
    nbuf = 2
    assert wpw % nbuf == 0 and wpw >= 2 * nbuf
    nchunk = wpw // nbuf
    mesh = plsc.VectorSubcoreMesh(core_axis_name="c", subcore_axis_name="s")

    @functools.partial(
        pl.kernel,
        mesh=mesh,
        out_type=jax.ShapeDtypeStruct((NC, n_pad, D), jnp.float32),
        scratch_types=[
            pltpu.VMEM((3 * nbuf, WIN), jnp.int32),
            pltpu.VMEM((3 * nbuf, WIN), jnp.int32),
        ] + [pltpu.VMEM((WIN, D), jnp.float32)] * nbuf + [
            pltpu.SemaphoreType.DMA,
            pltpu.SemaphoreType.DMA,
            pltpu.SemaphoreType.DMA,
            pltpu.SemaphoreType.DMA,
            pltpu.VMEM_SHARED((n_pad, D), jnp.float32),
        ],
    )
    def agg_kernel(xt_hbm, src_hbm, dst_hbm, out_hbm, is_ring, id_ring, *rest):
        bufs = rest[:nbuf]
        sem_g, sem_s, sem_x, sem_i, acc = rest[nbuf:]
        cid = lax.axis_index("c")
        sid = lax.axis_index("s")
        wid = sid * NC + cid
        base = sid * rows_per_sub
        init = pltpu.async_copy(xt_hbm.at[pl.ds(base, rows_per_sub)],
                                acc.at[pl.ds(base, rows_per_sub)], sem_i)

        def load_chunk(t, slot):
            s = pl.ds(slot * nbuf, nbuf)
            a = pltpu.async_copy(
                src_hbm.at[wid, pl.ds(t * nbuf, nbuf)], is_ring.at[s], sem_x)
            b = pltpu.async_copy(
                dst_hbm.at[wid, pl.ds(t * nbuf, nbuf)], id_ring.at[s], sem_x)
            return a, b

        a0, b0 = load_chunk(0, 0)
        a0.wait()
        b0.wait()
        load_chunk(1, 1)
        pltpu.async_copy(xt_hbm.at[is_ring.at[0]], bufs[0], sem_g)
        init.wait()
        plsc.subcore_barrier()

        def scat(row, b):
            return pltpu.make_async_copy(bufs[b], acc.at[id_ring.at[row]],
                                         sem_s)

        def body(t, _):
            slot = lax.rem(t, 3)
            nxt = lax.rem(t + 1, 3)
            prv = lax.rem(t + 2, 3)

            @pl.when(t + 1 < nchunk)
            def _wait_idx():
                s = pl.ds(nxt * nbuf, nbuf)
                e = pl.ds((t + 1) * nbuf, nbuf)
                pltpu.make_async_copy(
                    src_hbm.at[wid, e], is_ring.at[s], sem_x).wait()
                pltpu.make_async_copy(
                    dst_hbm.at[wid, e], id_ring.at[s], sem_x).wait()

            for b in range(nbuf):
                j = t * nbuf + b
                r = slot * nbuf + b
                pltpu.make_async_copy(
                    xt_hbm.at[is_ring.at[r]], bufs[b], sem_g).wait()
                pltpu.async_copy(bufs[b], acc.at[id_ring.at[r]], sem_s,
                                 add=True)

                @pl.when(j > 0)
                def _drain_prev():
                    pr = (prv * nbuf + 1) if b == 0 else (slot * nbuf)
                    scat(pr, 1 - b).wait()

                @pl.when(j + 1 < wpw)
                def _prefetch():
                    r2 = (slot * nbuf + 1) if b == 0 else (nxt * nbuf)
                    pltpu.async_copy(xt_hbm.at[is_ring.at[r2]], bufs[1 - b],
                                     sem_g)

            @pl.when(t + 2 < nchunk)
            def _next_idx():
                load_chunk(t + 2, prv)
            return _

        lax.fori_loop(0, nchunk, body, None)
        scat(lax.rem(nchunk - 1, 3) * nbuf + 1, 1).wait()
        plsc.subcore_barrier()
        pltpu.sync_copy(
            acc.at[pl.ds(base, rows_per_sub)],
            out_hbm.at[cid, pl.ds(base, rows_per_sub)],
        )

    return agg_kernel


def _tc_first(n_pad):
    grid = (n_pad // ROW_BLK,)

    def body(x_ref, w_ref, degp_ref, xt_ref, dis_ref):
        deg = degp_ref[:, 0:1] + degp_ref[:, 1:2] + 1.0
        dis = lax.rsqrt(deg)
        xw = jnp.dot(x_ref[...], w_ref[...], preferred_element_type=jnp.float32)
        xt_ref[...] = dis * xw
        dis_ref[...] = dis

    return pl.pallas_call(
        body,
        grid=grid,
        in_specs=[
            pl.BlockSpec((ROW_BLK, D), lambda i: (i, 0)),
            pl.BlockSpec((D, D), lambda i: (0, 0)),
            pl.BlockSpec((ROW_BLK, NC), lambda i: (i, 0)),
        ],
        out_specs=[
            pl.BlockSpec((ROW_BLK, D), lambda i: (i, 0)),
            pl.BlockSpec((ROW_BLK, 1), lambda i: (i, 0)),
        ],
        out_shape=[
            jax.ShapeDtypeStruct((n_pad, D), jnp.float32),
            jax.ShapeDtypeStruct((n_pad, 1), jnp.float32),
        ],
    )


def _tc_mid(n_pad, relu, want_pre):
    grid = (n_pad // ROW_BLK,)

    def body(acc_ref, xt_ref, dis_ref, b_ref, w_ref, *outs):
        s = acc_ref[0] + acc_ref[1] - xt_ref[...]
        pre = dis_ref[...] * s + b_ref[...]
        if relu:
            pre = jnp.maximum(pre, 0.0)
        nxt = dis_ref[...] * jnp.dot(
            pre, w_ref[...], preferred_element_type=jnp.float32)
        if want_pre:
            outs[0][...] = pre
            outs[1][...] = nxt
        else:
            outs[0][...] = nxt

    n_out = 2 if want_pre else 1
    return pl.pallas_call(
        body,
        grid=grid,
        in_specs=[
            pl.BlockSpec((NC, ROW_BLK, D), lambda i: (0, i, 0)),
            pl.BlockSpec((ROW_BLK, D), lambda i: (i, 0)),
            pl.BlockSpec((ROW_BLK, 1), lambda i: (i, 0)),
            pl.BlockSpec((1, D), lambda i: (0, 0)),
            pl.BlockSpec((D, D), lambda i: (0, 0)),
        ],
        out_specs=[pl.BlockSpec((ROW_BLK, D), lambda i: (i, 0))] * n_out,
        out_shape=[jax.ShapeDtypeStruct((n_pad, D), jnp.float32)] * n_out,
    )


def _tc_last(n_pad):
    grid = (n_pad // ROW_BLK,)

    def body(acc_ref, xt_ref, dis_ref, b_ref, xtemp_ref, out_ref):
        s = acc_ref[0] + acc_ref[1] - xt_ref[...]
        h = jnp.maximum(dis_ref[...] * s + b_ref[...], 0.0)
        out_ref[...] = h + xtemp_ref[...]

    return pl.pallas_call(
        body,
        grid=grid,
        in_specs=[
            pl.BlockSpec((NC, ROW_BLK, D), lambda i: (0, i, 0)),
            pl.BlockSpec((ROW_BLK, D), lambda i: (i, 0)),
            pl.BlockSpec((ROW_BLK, 1), lambda i: (i, 0)),
            pl.BlockSpec((1, D), lambda i: (0, 0)),
            pl.BlockSpec((ROW_BLK, D), lambda i: (i, 0)),
        ],
        out_specs=pl.BlockSpec((ROW_BLK, D), lambda i: (i, 0)),
        out_shape=jax.ShapeDtypeStruct((n_pad, D), jnp.float32),
    )


def kernel(x, edge_index, W0, b0, W1, b1, W2, b2):
    n = x.shape[0]
    e = edge_index.shape[1]
    n_pad = ((n + ROW_BLK - 1) // ROW_BLK) * ROW_BLK
    n_spare = n_pad - n
    grp = NW * WIN * 2
    e_pad = ((e + grp - 1) // grp) * grp
    wpw = e_pad // (NW * WIN)

    src = edge_index[0].astype(jnp.int32)
    dst = edge_index[1].astype(jnp.int32)
    pad_idx = n + (jnp.arange(e_pad - e, dtype=jnp.int32) % max(n_spare, 1))
    src_grp = jnp.concatenate([src, pad_idx]).reshape(NW, wpw, WIN)
    dst_grp = jnp.concatenate([dst, pad_idx]).reshape(NW, wpw, WIN)
    x_pad = jnp.zeros((n_pad, D), x.dtype).at[:n].set(x)

    degp = _sc_degree(n_pad, wpw)(dst_grp)
    degp_t = degp.T

    xt0, dis = _tc_first(n_pad)(x_pad, W0, degp_t)
    acc0 = _sc_aggregate(n_pad, wpw)(xt0, src_grp, dst_grp)
    x_temp, xt1 = _tc_mid(n_pad, relu=False, want_pre=True)(
        acc0, xt0, dis, b0.reshape(1, D), W1)
    acc1 = _sc_aggregate(n_pad, wpw)(xt1, src_grp, dst_grp)
    (xt2,) = _tc_mid(n_pad, relu=True, want_pre=False)(
        acc1, xt1, dis, b1.reshape(1, D), W2)
    acc2 = _sc_aggregate(n_pad, wpw)(xt2, src_grp, dst_grp)
    out = _tc_last(n_pad)(acc2, xt2, dis, b2.reshape(1, D), x_temp)
    return out[:n]


# speedup vs baseline: 1.1703x; 1.1703x over previous
import functools

import jax
import jax.numpy as jnp
from jax import lax
from jax.experimental import pallas as pl
from jax.experimental.pallas import tpu as pltpu
from jax.experimental.pallas import tpu_sc as plsc

D = 128
ROW_BLK = 2000
N_PAD_SC = 10240
NC = 2
NS = 16
NW = NC * NS
WIN = 128


def _sc_degree(n, n_pad, wpw):
    rows_per_sub = n_pad // NS
    out_per_sub = n // NS
    mesh = plsc.VectorSubcoreMesh(core_axis_name="c", subcore_axis_name="s")

    @functools.partial(
        pl.kernel,
        mesh=mesh,
        out_type=jax.ShapeDtypeStruct((NC, n_pad), jnp.float32),
        scratch_types=[
            pltpu.VMEM((wpw, WIN), jnp.int32),
            pltpu.VMEM((WIN,), jnp.float32),
            pltpu.VMEM((n_pad // NS,), jnp.float32),
            pltpu.VMEM_SHARED((n_pad,), jnp.float32),
        ],
    )
    def deg_kernel(dst_hbm, out_hbm, idx_v, ones_v, zeros_v, acc):
        cid = lax.axis_index("c")
        sid = lax.axis_index("s")
        wid = sid * NC + cid
        one = jnp.full((16,), 1.0, dtype=jnp.float32)
        zero = jnp.zeros((16,), dtype=jnp.float32)
        for i in range(WIN // 16):
            ones_v[pl.ds(i * 16, 16)] = one

        def fill_zero(i, _):
            zeros_v[pl.ds(i * 16, 16)] = zero
            return _

        lax.fori_loop(0, rows_per_sub // 16, fill_zero, None)
        pltpu.sync_copy(zeros_v, acc.at[pl.ds(sid * rows_per_sub, rows_per_sub)])
        pltpu.sync_copy(dst_hbm.at[wid], idx_v)
        plsc.subcore_barrier()

        def body(j, _):
            pltpu.sync_copy(ones_v, acc.at[idx_v.at[j]], add=True)
            return _

        lax.fori_loop(0, wpw, body, None)
        plsc.subcore_barrier()
        pltpu.sync_copy(
            acc.at[pl.ds(sid * rows_per_sub, rows_per_sub)],
            out_hbm.at[cid, pl.ds(sid * rows_per_sub, rows_per_sub)],
        )

    return deg_kernel


def _sc_aggregate(n, n_pad, wpw):
    row_chunk = (n // NS) & ~7
    row_rem = n - NS * row_chunk
    nbuf = 2
    assert wpw % nbuf == 0 and wpw >= 2 * nbuf
    nchunk = wpw // nbuf
    mesh = plsc.VectorSubcoreMesh(core_axis_name="c", subcore_axis_name="s")

    @functools.partial(
        pl.kernel,
        mesh=mesh,
        out_type=[jax.ShapeDtypeStruct((n, D), jnp.float32)] * NC,
        scratch_types=[
            pltpu.VMEM((2 * nbuf, WIN), jnp.int32),
            pltpu.VMEM((2 * nbuf, WIN), jnp.int32),
        ] + [pltpu.VMEM((WIN, D), jnp.float32)] * nbuf + [
            pltpu.SemaphoreType.DMA,
            pltpu.SemaphoreType.DMA,
            pltpu.SemaphoreType.DMA,
            pltpu.VMEM_SHARED((n_pad, D), jnp.float32),
        ],
    )
    def agg_kernel(xt_hbm, src_hbm, dst_hbm, out0_hbm, out1_hbm,
                   is_ring, id_ring, *rest):
        bufs = rest[:nbuf]
        sem_g, sem_x, sem_i, acc = rest[nbuf:]
        cid = lax.axis_index("c")
        sid = lax.axis_index("s")
        wid = sid * NC + cid
        base = sid * row_chunk
        init = pltpu.async_copy(xt_hbm.at[pl.ds(base, row_chunk)],
                                acc.at[pl.ds(base, row_chunk)], sem_i)
        if row_rem:
            @pl.when(sid == NS - 1)
            def _init_tail():
                pltpu.sync_copy(xt_hbm.at[pl.ds(NS * row_chunk, row_rem)],
                                acc.at[pl.ds(NS * row_chunk, row_rem)])

        def load_chunk(t, half):
            s = pl.ds(half * nbuf, nbuf)
            a = pltpu.async_copy(
                src_hbm.at[wid, pl.ds(t * nbuf, nbuf)], is_ring.at[s], sem_x)
            b = pltpu.async_copy(
                dst_hbm.at[wid, pl.ds(t * nbuf, nbuf)], id_ring.at[s], sem_x)
            return a, b

        a0, b0 = load_chunk(0, 0)
        a0.wait()
        b0.wait()
        load_chunk(1, 1)
        for b in range(nbuf):
            pltpu.async_copy(xt_hbm.at[is_ring.at[b]], bufs[b], sem_g)
        init.wait()
        plsc.subcore_barrier()

        def body(t, _):
            half = lax.rem(t, 2)
            nxt_half = lax.rem(t + 1, 2)

            @pl.when(t + 1 < nchunk)
            def _wait_idx():
                s = pl.ds(nxt_half * nbuf, nbuf)
                e = pl.ds((t + 1) * nbuf, nbuf)
                pltpu.make_async_copy(
                    src_hbm.at[wid, e], is_ring.at[s], sem_x).wait()
                pltpu.make_async_copy(
                    dst_hbm.at[wid, e], id_ring.at[s], sem_x).wait()

            for b in range(nbuf):
                j = t * nbuf + b
                r = half * nbuf + b
                pltpu.make_async_copy(
                    xt_hbm.at[is_ring.at[r]], bufs[b], sem_g).wait()
                pltpu.sync_copy(bufs[b], acc.at[id_ring.at[r]], add=True)

                @pl.when(j + nbuf < wpw)
                def _prefetch():
                    r2 = nxt_half * nbuf + b
                    pltpu.async_copy(xt_hbm.at[is_ring.at[r2]], bufs[b], sem_g)

            @pl.when(t + 2 < nchunk)
            def _next_idx():
                load_chunk(t + 2, half)
            return _

        lax.fori_loop(0, nchunk, body, None)
        plsc.subcore_barrier()

        def emit(out_hbm):
            pltpu.sync_copy(acc.at[pl.ds(base, row_chunk)],
                            out_hbm.at[pl.ds(base, row_chunk)])
            if row_rem:
                @pl.when(sid == NS - 1)
                def _tail():
                    pltpu.sync_copy(acc.at[pl.ds(NS * row_chunk, row_rem)],
                                    out_hbm.at[pl.ds(NS * row_chunk, row_rem)])

        @pl.when(cid == 0)
        def _out0():
            emit(out0_hbm)

        @pl.when(cid == 1)
        def _out1():
            emit(out1_hbm)

    return agg_kernel


def _tc_first(n):
    grid = (n // ROW_BLK,)

    def body(x_ref, w_ref, degp_ref, xt_ref, dis_ref):
        deg = degp_ref[:, 0:1] + degp_ref[:, 1:2] + 1.0
        dis = lax.rsqrt(deg)
        xw = jnp.dot(x_ref[...], w_ref[...], preferred_element_type=jnp.float32)
        xt_ref[...] = dis * xw
        dis_ref[...] = dis

    return pl.pallas_call(
        body,
        grid=grid,
        in_specs=[
            pl.BlockSpec((ROW_BLK, D), lambda i: (i, 0)),
            pl.BlockSpec((D, D), lambda i: (0, 0)),
            pl.BlockSpec((ROW_BLK, NC), lambda i: (i, 0)),
        ],
        out_specs=[
            pl.BlockSpec((ROW_BLK, D), lambda i: (i, 0)),
            pl.BlockSpec((ROW_BLK, 1), lambda i: (i, 0)),
        ],
        out_shape=[
            jax.ShapeDtypeStruct((n, D), jnp.float32),
            jax.ShapeDtypeStruct((n, 1), jnp.float32),
        ],
    )


def _tc_mid(n, relu, want_pre):
    grid = (n // ROW_BLK,)

    def body(a0_ref, a1_ref, xt_ref, dis_ref, b_ref, w_ref, *outs):
        s = a0_ref[...] + a1_ref[...] - xt_ref[...]
        pre = dis_ref[...] * s + b_ref[...]
        if relu:
            pre = jnp.maximum(pre, 0.0)
        nxt = dis_ref[...] * jnp.dot(
            pre, w_ref[...], preferred_element_type=jnp.float32)
        if want_pre:
            outs[0][...] = pre
            outs[1][...] = nxt
        else:
            outs[0][...] = nxt

    n_out = 2 if want_pre else 1
    return pl.pallas_call(
        body,
        grid=grid,
        in_specs=[
            pl.BlockSpec((ROW_BLK, D), lambda i: (i, 0)),
            pl.BlockSpec((ROW_BLK, D), lambda i: (i, 0)),
            pl.BlockSpec((ROW_BLK, D), lambda i: (i, 0)),
            pl.BlockSpec((ROW_BLK, 1), lambda i: (i, 0)),
            pl.BlockSpec((1, D), lambda i: (0, 0)),
            pl.BlockSpec((D, D), lambda i: (0, 0)),
        ],
        out_specs=[pl.BlockSpec((ROW_BLK, D), lambda i: (i, 0))] * n_out,
        out_shape=[jax.ShapeDtypeStruct((n, D), jnp.float32)] * n_out,
    )


def _tc_last(n):
    grid = (n // ROW_BLK,)

    def body(a0_ref, a1_ref, xt_ref, dis_ref, b_ref, xtemp_ref, out_ref):
        s = a0_ref[...] + a1_ref[...] - xt_ref[...]
        h = jnp.maximum(dis_ref[...] * s + b_ref[...], 0.0)
        out_ref[...] = h + xtemp_ref[...]

    return pl.pallas_call(
        body,
        grid=grid,
        in_specs=[
            pl.BlockSpec((ROW_BLK, D), lambda i: (i, 0)),
            pl.BlockSpec((ROW_BLK, D), lambda i: (i, 0)),
            pl.BlockSpec((ROW_BLK, D), lambda i: (i, 0)),
            pl.BlockSpec((ROW_BLK, 1), lambda i: (i, 0)),
            pl.BlockSpec((1, D), lambda i: (0, 0)),
            pl.BlockSpec((ROW_BLK, D), lambda i: (i, 0)),
        ],
        out_specs=pl.BlockSpec((ROW_BLK, D), lambda i: (i, 0)),
        out_shape=jax.ShapeDtypeStruct((n, D), jnp.float32),
    )


def kernel(x, edge_index, W0, b0, W1, b1, W2, b2):
    n = x.shape[0]
    e = edge_index.shape[1]
    n_pad = N_PAD_SC
    assert n % ROW_BLK == 0 and n % NS == 0 and n < n_pad and n_pad % NS == 0
    grp = NW * WIN * 2
    e_pad = ((e + grp - 1) // grp) * grp
    wpw = e_pad // (NW * WIN)

    src = edge_index[0].astype(jnp.int32)
    dst = edge_index[1].astype(jnp.int32)
    npe = e_pad - e
    pad_src = jnp.arange(npe, dtype=jnp.int32) % n
    pad_dst = n + (jnp.arange(npe, dtype=jnp.int32) % (n_pad - n))
    src_grp = jnp.concatenate([src, pad_src]).reshape(NW, wpw, WIN)
    dst_grp = jnp.concatenate([dst, pad_dst]).reshape(NW, wpw, WIN)

    degp = _sc_degree(n, n_pad, wpw)(dst_grp)
    degp_t = degp.T[:n]

    agg = _sc_aggregate(n, n_pad, wpw)
    xt0, dis = _tc_first(n)(x, W0, degp_t)
    a0, a1 = agg(xt0, src_grp, dst_grp)
    x_temp, xt1 = _tc_mid(n, relu=False, want_pre=True)(
        a0, a1, xt0, dis, b0.reshape(1, D), W1)
    a0, a1 = agg(xt1, src_grp, dst_grp)
    (xt2,) = _tc_mid(n, relu=True, want_pre=False)(
        a0, a1, xt1, dis, b1.reshape(1, D), W2)
    a0, a1 = agg(xt2, src_grp, dst_grp)
    return _tc_last(n)(a0, a1, xt2, dis, b2.reshape(1, D), x_temp)


# speedup vs baseline: 1.2366x; 1.0567x over previous
import functools

import jax
import jax.numpy as jnp
from jax import lax
from jax.experimental import pallas as pl
from jax.experimental.pallas import tpu as pltpu
from jax.experimental.pallas import tpu_sc as plsc

D = 128
ROW_BLK = 2000
N_PAD_SC = 10240
NC = 2
NS = 16
NW = NC * NS
WIN = 128


def _worker_span(wid, tw):
    q4 = tw // 4
    base = q4 // NW
    extra = q4 - NW * base
    w0 = 4 * (wid * base + jnp.minimum(wid, extra))
    nwin = 4 * (base + (wid < extra).astype(jnp.int32))
    return w0, nwin


def _sc_degree(n, n_pad, tw):
    rows_per_sub = n_pad // NS
    mesh = plsc.VectorSubcoreMesh(core_axis_name="c", subcore_axis_name="s")

    @functools.partial(
        pl.kernel,
        mesh=mesh,
        out_type=jax.ShapeDtypeStruct((NC, n_pad), jnp.float32),
        scratch_types=[
            pltpu.VMEM((4, 2, WIN), jnp.int32),
            pltpu.VMEM((WIN,), jnp.float32),
            pltpu.VMEM((n_pad // NS,), jnp.float32),
            pltpu.SemaphoreType.DMA,
            pltpu.VMEM_SHARED((n_pad,), jnp.float32),
        ],
    )
    def deg_kernel(edges_hbm, out_hbm, exi, ones_v, zeros_v, sem_x, acc):
        cid = lax.axis_index("c")
        sid = lax.axis_index("s")
        wid = sid * NC + cid
        w0, nwin = _worker_span(wid, tw)

        def load_w(g, slot):
            pltpu.async_copy(
                edges_hbm.at[:, pl.ds((w0 + g) * WIN, WIN)], exi.at[slot],
                sem_x)

        def wait_w(g, slot):
            pltpu.make_async_copy(
                edges_hbm.at[:, pl.ds((w0 + g) * WIN, WIN)], exi.at[slot],
                sem_x).wait()

        one = jnp.full((16,), 1.0, dtype=jnp.float32)
        zero = jnp.zeros((16,), dtype=jnp.float32)
        for i in range(WIN // 16):
            ones_v[pl.ds(i * 16, 16)] = one

        def fill_zero(i, _):
            zeros_v[pl.ds(i * 16, 16)] = zero
            return _

        lax.fori_loop(0, rows_per_sub // 16, fill_zero, None)
        for k in range(4):
            load_w(k, k)
        pltpu.sync_copy(zeros_v, acc.at[pl.ds(sid * rows_per_sub, rows_per_sub)])
        plsc.subcore_barrier()

        def body(t, _):
            for k in range(4):
                j = t * 4 + k
                wait_w(j, k)
                pltpu.sync_copy(ones_v, acc.at[exi.at[k, 1]], add=True)

                @pl.when(j + 4 < nwin)
                def _pref():
                    load_w(j + 4, k)
            return _

        lax.fori_loop(0, nwin // 4, body, None)
        plsc.subcore_barrier()
        pltpu.sync_copy(
            acc.at[pl.ds(sid * rows_per_sub, rows_per_sub)],
            out_hbm.at[cid, pl.ds(sid * rows_per_sub, rows_per_sub)],
        )

    return deg_kernel


def _sc_aggregate(n, n_pad, tw):
    row_chunk = (n // NS) & ~7
    row_rem = n - NS * row_chunk
    mesh = plsc.VectorSubcoreMesh(core_axis_name="c", subcore_axis_name="s")

    @functools.partial(
        pl.kernel,
        mesh=mesh,
        out_type=[jax.ShapeDtypeStruct((n, D), jnp.float32)] * NC,
        scratch_types=[
            pltpu.VMEM((4, 2, WIN), jnp.int32),
            pltpu.VMEM((WIN, D), jnp.float32),
            pltpu.VMEM((WIN, D), jnp.float32),
            pltpu.SemaphoreType.DMA,
            pltpu.SemaphoreType.DMA,
            pltpu.SemaphoreType.DMA,
            pltpu.VMEM_SHARED((n_pad, D), jnp.float32),
        ],
    )
    def agg_kernel(xt_hbm, edges_hbm, out0_hbm, out1_hbm,
                   exi, buf0, buf1, sem_g, sem_x, sem_i, acc):
        bufs = (buf0, buf1)
        cid = lax.axis_index("c")
        sid = lax.axis_index("s")
        wid = sid * NC + cid
        w0, nwin = _worker_span(wid, tw)
        base = sid * row_chunk
        init = pltpu.async_copy(xt_hbm.at[pl.ds(base, row_chunk)],
                                acc.at[pl.ds(base, row_chunk)], sem_i)
        if row_rem:
            @pl.when(sid == NS - 1)
            def _init_tail():
                pltpu.sync_copy(xt_hbm.at[pl.ds(NS * row_chunk, row_rem)],
                                acc.at[pl.ds(NS * row_chunk, row_rem)])

        def load_w(g, slot):
            pltpu.async_copy(
                edges_hbm.at[:, pl.ds((w0 + g) * WIN, WIN)], exi.at[slot],
                sem_x)

        def wait_w(g, slot):
            pltpu.make_async_copy(
                edges_hbm.at[:, pl.ds((w0 + g) * WIN, WIN)], exi.at[slot],
                sem_x).wait()

        def gather(slot, b):
            pltpu.async_copy(xt_hbm.at[exi.at[slot, 0]], bufs[b], sem_g)

        def wait_gather(slot, b):
            pltpu.make_async_copy(xt_hbm.at[exi.at[slot, 0]], bufs[b],
                                  sem_g).wait()

        load_w(0, 0)
        load_w(1, 1)
        wait_w(0, 0)
        gather(0, 0)
        load_w(2, 2)
        wait_w(1, 1)
        gather(1, 1)
        load_w(3, 3)
        init.wait()
        plsc.subcore_barrier()

        def body(t, _):
            for k in range(4):
                j = t * 4 + k
                b = k % 2
                wait_gather(k, b)
                pltpu.sync_copy(bufs[b], acc.at[exi.at[k, 1]], add=True)

                @pl.when(j + 4 < nwin)
                def _pref_idx():
                    load_w(j + 4, k)

                @pl.when(j + 2 < nwin)
                def _pref_gather():
                    k2 = (k + 2) % 4
                    wait_w(j + 2, k2)
                    gather(k2, b)
            return _

        lax.fori_loop(0, nwin // 4, body, None)
        plsc.subcore_barrier()

        def emit(out_hbm):
            pltpu.sync_copy(acc.at[pl.ds(base, row_chunk)],
                            out_hbm.at[pl.ds(base, row_chunk)])
            if row_rem:
                @pl.when(sid == NS - 1)
                def _tail():
                    pltpu.sync_copy(acc.at[pl.ds(NS * row_chunk, row_rem)],
                                    out_hbm.at[pl.ds(NS * row_chunk, row_rem)])

        @pl.when(cid == 0)
        def _out0():
            emit(out0_hbm)

        @pl.when(cid == 1)
        def _out1():
            emit(out1_hbm)

    return agg_kernel


def _tc_first(n):
    grid = (n // ROW_BLK,)

    def body(x_ref, w_ref, degp_ref, xt_ref, dis_ref):
        deg = degp_ref[:, 0:1] + degp_ref[:, 1:2] + 1.0
        dis = lax.rsqrt(deg)
        xw = jnp.dot(x_ref[...], w_ref[...], preferred_element_type=jnp.float32)
        xt_ref[...] = dis * xw
        dis_ref[...] = dis

    return pl.pallas_call(
        body,
        grid=grid,
        in_specs=[
            pl.BlockSpec((ROW_BLK, D), lambda i: (i, 0)),
            pl.BlockSpec((D, D), lambda i: (0, 0)),
            pl.BlockSpec((ROW_BLK, NC), lambda i: (i, 0)),
        ],
        out_specs=[
            pl.BlockSpec((ROW_BLK, D), lambda i: (i, 0)),
            pl.BlockSpec((ROW_BLK, 1), lambda i: (i, 0)),
        ],
        out_shape=[
            jax.ShapeDtypeStruct((n, D), jnp.float32),
            jax.ShapeDtypeStruct((n, 1), jnp.float32),
        ],
    )


def _tc_mid(n, relu, want_pre):
    grid = (n // ROW_BLK,)

    def body(a0_ref, a1_ref, xt_ref, dis_ref, b_ref, w_ref, *outs):
        s = a0_ref[...] + a1_ref[...] - xt_ref[...]
        pre = dis_ref[...] * s + b_ref[...]
        if relu:
            pre = jnp.maximum(pre, 0.0)
        nxt = dis_ref[...] * jnp.dot(
            pre, w_ref[...], preferred_element_type=jnp.float32)
        if want_pre:
            outs[0][...] = pre
            outs[1][...] = nxt
        else:
            outs[0][...] = nxt

    n_out = 2 if want_pre else 1
    return pl.pallas_call(
        body,
        grid=grid,
        in_specs=[
            pl.BlockSpec((ROW_BLK, D), lambda i: (i, 0)),
            pl.BlockSpec((ROW_BLK, D), lambda i: (i, 0)),
            pl.BlockSpec((ROW_BLK, D), lambda i: (i, 0)),
            pl.BlockSpec((ROW_BLK, 1), lambda i: (i, 0)),
            pl.BlockSpec((1, D), lambda i: (0, 0)),
            pl.BlockSpec((D, D), lambda i: (0, 0)),
        ],
        out_specs=[pl.BlockSpec((ROW_BLK, D), lambda i: (i, 0))] * n_out,
        out_shape=[jax.ShapeDtypeStruct((n, D), jnp.float32)] * n_out,
    )


def _tc_last(n):
    grid = (n // ROW_BLK,)

    def body(a0_ref, a1_ref, xt_ref, dis_ref, b_ref, xtemp_ref, out_ref):
        s = a0_ref[...] + a1_ref[...] - xt_ref[...]
        h = jnp.maximum(dis_ref[...] * s + b_ref[...], 0.0)
        out_ref[...] = h + xtemp_ref[...]

    return pl.pallas_call(
        body,
        grid=grid,
        in_specs=[
            pl.BlockSpec((ROW_BLK, D), lambda i: (i, 0)),
            pl.BlockSpec((ROW_BLK, D), lambda i: (i, 0)),
            pl.BlockSpec((ROW_BLK, D), lambda i: (i, 0)),
            pl.BlockSpec((ROW_BLK, 1), lambda i: (i, 0)),
            pl.BlockSpec((1, D), lambda i: (0, 0)),
            pl.BlockSpec((ROW_BLK, D), lambda i: (i, 0)),
        ],
        out_specs=pl.BlockSpec((ROW_BLK, D), lambda i: (i, 0)),
        out_shape=jax.ShapeDtypeStruct((n, D), jnp.float32),
    )


def kernel(x, edge_index, W0, b0, W1, b1, W2, b2):
    n = x.shape[0]
    e = edge_index.shape[1]
    n_pad = N_PAD_SC
    assert n % ROW_BLK == 0 and n % NS == 0 and n < n_pad and n_pad % NS == 0
    assert e % (4 * WIN) == 0
    tw = e // WIN
    edges = edge_index.astype(jnp.int32)

    degp = _sc_degree(n, n_pad, tw)(edges)
    degp_t = degp.T

    agg = _sc_aggregate(n, n_pad, tw)
    xt0, dis = _tc_first(n)(x, W0, degp_t)
    a0, a1 = agg(xt0, edges)
    x_temp, xt1 = _tc_mid(n, relu=False, want_pre=True)(
        a0, a1, xt0, dis, b0.reshape(1, D), W1)
    a0, a1 = agg(xt1, edges)
    (xt2,) = _tc_mid(n, relu=True, want_pre=False)(
        a0, a1, xt1, dis, b1.reshape(1, D), W2)
    a0, a1 = agg(xt2, edges)
    return _tc_last(n)(a0, a1, xt2, dis, b2.reshape(1, D), x_temp)
